# Initial kernel scaffold; baseline (speedup 1.0000x reference)
import jax
import jax.numpy as jnp
from jax.experimental import pallas as pl


def kernel(input, adj, M, W, a):
    raise NotImplementedError("write your pallas kernel here")


# speedup vs baseline: 2.2894x; 2.2894x over previous
import functools

import jax
import jax.numpy as jnp
from jax.experimental import pallas as pl

_THR = 0.05
_ALPHA = 0.2


def _pick_block(n, candidates):
    for b in candidates:
        if n % b == 0 and b % 8 == 0:
            return b
    return n


def _leaky(x):
    return jnp.where(x > 0, x, _ALPHA * x)


def _prologue_body(x_ref, w_ref, ap_ref, h_ref, st_ref):
    h = jnp.dot(x_ref[...], w_ref[...], preferred_element_type=jnp.float32)
    h_ref[...] = h
    s12 = jnp.dot(h, ap_ref[...], preferred_element_type=jnp.float32)
    rs = jnp.sum(h, axis=1, keepdims=True)
    zero = jnp.zeros_like(rs)
    st_ref[...] = jnp.concatenate([s12, rs, zero], axis=1)


def _attn_body(adj_ref, s1_ref, aux_ref, h_ref, out_ref, *, block_b):
    i = pl.program_id(0)
    adjb = adj_ref[...]
    s1 = s1_ref[...]
    s2 = aux_ref[0:1, :]
    vld = aux_ref[1:2, :]
    mask = (adjb > 0.0) & (vld > 0.0)
    maskf = jnp.where(mask, 1.0, 0.0)
    big_neg = jnp.float32(-3.0e38)
    m2 = jnp.max(jnp.where(mask, s2, big_neg), axis=1, keepdims=True)
    big_l = jnp.sum(maskf, axis=1, keepdims=True)
    m2 = jnp.where(big_l > 0, m2, 0.0)
    emax = _leaky(s1 + m2)
    e = _leaky(s1 + s2)
    p = jnp.where(mask, jnp.exp(e - emax), 0.0)
    z = jnp.sum(p, axis=1, keepdims=True)
    zs = jnp.where(z > 0, z, 1.0)
    att = p / zs
    top = jnp.max(att, axis=1, keepdims=True)
    w = jnp.where(att >= _THR, att, 0.0) * big_l
    w = jnp.where(top > _THR, w, 0.0)
    hp = jnp.dot(w, h_ref[...], preferred_element_type=jnp.float32)
    hb = h_ref[pl.ds(i * block_b, block_b), :]
    y = hb + hp
    out_ref[...] = jnp.where(y > 0, y, jnp.exp(y) - 1.0)


def kernel(input, adj, M, W, a):
    x = jnp.asarray(input, jnp.float32)
    n, d_in = x.shape
    d_out = W.shape[1]
    a_pair = jnp.concatenate([a[:d_out], a[d_out:]], axis=1)

    bp = _pick_block(n, (2000, 1000, 400, 200, 80, 40, 16, 8))
    h, stats = pl.pallas_call(
        _prologue_body,
        grid=(n // bp,),
        in_specs=[
            pl.BlockSpec((bp, d_in), lambda i: (i, 0)),
            pl.BlockSpec((d_in, d_out), lambda i: (0, 0)),
            pl.BlockSpec((d_out, 2), lambda i: (0, 0)),
        ],
        out_specs=[
            pl.BlockSpec((bp, d_out), lambda i: (i, 0)),
            pl.BlockSpec((bp, 4), lambda i: (i, 0)),
        ],
        out_shape=[
            jax.ShapeDtypeStruct((n, d_out), jnp.float32),
            jax.ShapeDtypeStruct((n, 4), jnp.float32),
        ],
    )(x, W, a_pair)

    s1c = stats[:, 0:1]
    s2r = jnp.reshape(stats[:, 1], (1, n))
    vldr = (jnp.reshape(stats[:, 2], (1, n)) != 0.0).astype(jnp.float32)
    aux = jnp.concatenate([s2r, vldr], axis=0)

    b = _pick_block(n, (200, 80, 40, 16, 8))
    out = pl.pallas_call(
        functools.partial(_attn_body, block_b=b),
        grid=(n // b,),
        in_specs=[
            pl.BlockSpec((b, n), lambda i: (i, 0)),
            pl.BlockSpec((b, 1), lambda i: (i, 0)),
            pl.BlockSpec((2, n), lambda i: (0, 0)),
            pl.BlockSpec((n, d_out), lambda i: (0, 0)),
        ],
        out_specs=pl.BlockSpec((b, d_out), lambda i: (i, 0)),
        out_shape=jax.ShapeDtypeStruct((n, d_out), jnp.float32),
    )(adj, s1c, aux, h)
    return out


# speedup vs baseline: 3.0026x; 1.3115x over previous
import functools

import jax
import jax.numpy as jnp
from jax.experimental import pallas as pl

_THR = 0.05
_ALPHA = 0.2
_BIG_NEG = -3.0e38


def _pick_block(n, candidates):
    for b in candidates:
        if n % b == 0 and b % 8 == 0:
            return b
    return n


def _prologue_body(x_ref, w_ref, ap_ref, h_ref, st_ref, mx_ref):
    i = pl.program_id(0)
    h = jnp.dot(x_ref[...], w_ref[...], preferred_element_type=jnp.float32)
    h_ref[...] = h
    s12 = jnp.dot(h, ap_ref[...], preferred_element_type=jnp.float32)
    rs = jnp.sum(h, axis=1, keepdims=True)
    s2m = jnp.where(rs != 0.0, s12[:, 1:2], _BIG_NEG)
    st_ref[...] = jnp.concatenate([s12, rs, s2m], axis=1)

    @pl.when(i == 0)
    def _init():
        mx_ref[...] = jnp.full((1, 1), _BIG_NEG, jnp.float32)

    mx_ref[...] = jnp.maximum(mx_ref[...], jnp.max(s2m).reshape(1, 1))


def _attn_body(adj_ref, s1_ref, aux_ref, mx_ref, h_ref, out_ref, *, block_b):
    i = pl.program_id(0)
    adjb = adj_ref[...]
    s1 = s1_ref[...]
    s2m = aux_ref[0:1, :]
    vldf = aux_ref[1:2, :]
    c0 = s1 + mx_ref[...]
    emaxc = jnp.maximum(c0, _ALPHA * c0)
    m0 = adjb > 0.0
    maskf = jnp.where(m0, vldf, 0.0)
    big_l = jnp.sum(maskf, axis=1, keepdims=True)
    pre = s1 + s2m
    e = jnp.maximum(pre, _ALPHA * pre)
    x = jnp.exp(e - emaxc)
    p = jnp.where(m0, x, 0.0)
    z = jnp.sum(p, axis=1, keepdims=True)
    pmax = jnp.max(p, axis=1, keepdims=True)
    zs = jnp.where(z > 0, z, 1.0)
    top = pmax / zs
    wl = jnp.where(top > _THR, big_l / zs, 0.0)
    thr2 = _THR * zs
    w = jnp.where(p >= thr2, p, 0.0) * wl
    hp = jnp.dot(w, h_ref[...], preferred_element_type=jnp.float32)
    hb = h_ref[pl.ds(i * block_b, block_b), :]
    y = hb + hp
    out_ref[...] = jnp.where(y > 0, y, jnp.exp(y) - 1.0)


def kernel(input, adj, M, W, a):
    x = jnp.asarray(input, jnp.float32)
    n, d_in = x.shape
    d_out = W.shape[1]
    a_pair = jnp.concatenate([a[:d_out], a[d_out:]], axis=1)

    bp = _pick_block(n, (2000, 1000, 400, 200, 80, 40, 16, 8))
    h, stats, mx = pl.pallas_call(
        _prologue_body,
        grid=(n // bp,),
        in_specs=[
            pl.BlockSpec((bp, d_in), lambda i: (i, 0)),
            pl.BlockSpec((d_in, d_out), lambda i: (0, 0)),
            pl.BlockSpec((d_out, 2), lambda i: (0, 0)),
        ],
        out_specs=[
            pl.BlockSpec((bp, d_out), lambda i: (i, 0)),
            pl.BlockSpec((bp, 4), lambda i: (i, 0)),
            pl.BlockSpec((1, 1), lambda i: (0, 0)),
        ],
        out_shape=[
            jax.ShapeDtypeStruct((n, d_out), jnp.float32),
            jax.ShapeDtypeStruct((n, 4), jnp.float32),
            jax.ShapeDtypeStruct((1, 1), jnp.float32),
        ],
    )(x, W, a_pair)

    s1c = stats[:, 0:1]
    s2mr = jnp.reshape(stats[:, 3], (1, n))
    vldr = (jnp.reshape(stats[:, 2], (1, n)) != 0.0).astype(jnp.float32)
    aux = jnp.concatenate([s2mr, vldr], axis=0)

    b = _pick_block(n, (200, 80, 40, 16, 8))
    out = pl.pallas_call(
        functools.partial(_attn_body, block_b=b),
        grid=(n // b,),
        in_specs=[
            pl.BlockSpec((b, n), lambda i: (i, 0)),
            pl.BlockSpec((b, 1), lambda i: (i, 0)),
            pl.BlockSpec((2, n), lambda i: (0, 0)),
            pl.BlockSpec((1, 1), lambda i: (0, 0)),
            pl.BlockSpec((n, d_out), lambda i: (0, 0)),
        ],
        out_specs=pl.BlockSpec((b, d_out), lambda i: (i, 0)),
        out_shape=jax.ShapeDtypeStruct((n, d_out), jnp.float32),
    )(adj, s1c, aux, mx, h)
    return out


# speedup vs baseline: 3.2976x; 1.0983x over previous
import functools

import jax
import jax.numpy as jnp
from jax.experimental import pallas as pl

_THR = 0.05
_ALPHA = 0.2
_BIG_NEG = -3.0e38
_LOG2E = 1.4426950408889634


def _pick_block(n, candidates):
    for b in candidates:
        if n % b == 0 and b % 8 == 0:
            return b
    return n


def _prologue_body(x_ref, w_ref, ap_ref, h_ref, st_ref, mx_ref):
    i = pl.program_id(0)
    h = jnp.dot(x_ref[...], w_ref[...], preferred_element_type=jnp.float32)
    h_ref[...] = h
    s12 = jnp.dot(h, ap_ref[...], preferred_element_type=jnp.float32)
    rs = jnp.sum(h, axis=1, keepdims=True)
    s2m = jnp.where(rs != 0.0, s12[:, 1:2], _BIG_NEG)
    s2l = s2m * jnp.float32(_LOG2E)
    s2al = s2m * jnp.float32(_ALPHA * _LOG2E)
    st_ref[...] = jnp.concatenate([s12[:, 0:1], rs, s2l, s2al], axis=1)

    @pl.when(i == 0)
    def _init():
        mx_ref[...] = jnp.full((1, 1), _BIG_NEG, jnp.float32)

    mx_ref[...] = jnp.maximum(mx_ref[...], jnp.max(s2m).reshape(1, 1))


def _attn_body(adj_ref, s1_ref, aux_ref, mx_ref, h_ref, out_ref, *, block_b):
    i = pl.program_id(0)
    adjb = adj_ref[...]
    s1 = s1_ref[...]
    s2l = aux_ref[0:1, :]
    s2al = aux_ref[1:2, :]
    vldf = aux_ref[2:3, :]
    c0 = s1 + mx_ref[...]
    emaxc = jnp.maximum(c0, _ALPHA * c0)
    d1 = (s1 - emaxc) * jnp.float32(_LOG2E)
    d2 = (_ALPHA * s1 - emaxc) * jnp.float32(_LOG2E)
    m0 = adjb > 0.0
    maskf = jnp.where(m0, vldf, 0.0)
    big_l = jnp.sum(maskf, axis=1, keepdims=True)
    t = jnp.maximum(s2l + d1, s2al + d2)
    x = jnp.exp2(t)
    p = jnp.where(m0, x, 0.0)
    z = jnp.sum(p, axis=1, keepdims=True)
    pmax = jnp.max(p, axis=1, keepdims=True)
    zs = jnp.where(z > 0, z, 1.0)
    top = pmax / zs
    wl = jnp.where(top > _THR, big_l / zs, 0.0)
    thr2 = _THR * zs
    w = jnp.where(p >= thr2, p, 0.0) * wl
    hp = jnp.dot(w, h_ref[...], preferred_element_type=jnp.float32)
    hb = h_ref[pl.ds(i * block_b, block_b), :]
    y = hb + hp
    out_ref[...] = jnp.where(y > 0, y, jnp.exp(y) - 1.0)


def kernel(input, adj, M, W, a):
    x = jnp.asarray(input, jnp.float32)
    n, d_in = x.shape
    d_out = W.shape[1]
    a_pair = jnp.concatenate([a[:d_out], a[d_out:]], axis=1)

    bp = _pick_block(n, (2000, 1000, 400, 200, 80, 40, 16, 8))
    h, stats, mx = pl.pallas_call(
        _prologue_body,
        grid=(n // bp,),
        in_specs=[
            pl.BlockSpec((bp, d_in), lambda i: (i, 0)),
            pl.BlockSpec((d_in, d_out), lambda i: (0, 0)),
            pl.BlockSpec((d_out, 2), lambda i: (0, 0)),
        ],
        out_specs=[
            pl.BlockSpec((bp, d_out), lambda i: (i, 0)),
            pl.BlockSpec((bp, 4), lambda i: (i, 0)),
            pl.BlockSpec((1, 1), lambda i: (0, 0)),
        ],
        out_shape=[
            jax.ShapeDtypeStruct((n, d_out), jnp.float32),
            jax.ShapeDtypeStruct((n, 4), jnp.float32),
            jax.ShapeDtypeStruct((1, 1), jnp.float32),
        ],
    )(x, W, a_pair)

    s1c = stats[:, 0:1]
    s2lr = jnp.reshape(stats[:, 2], (1, n))
    s2alr = jnp.reshape(stats[:, 3], (1, n))
    vldr = (jnp.reshape(stats[:, 1], (1, n)) != 0.0).astype(jnp.float32)
    aux = jnp.concatenate([s2lr, s2alr, vldr], axis=0)

    b = _pick_block(n, (200, 80, 40, 16, 8))
    out = pl.pallas_call(
        functools.partial(_attn_body, block_b=b),
        grid=(n // b,),
        in_specs=[
            pl.BlockSpec((b, n), lambda i: (i, 0)),
            pl.BlockSpec((b, 1), lambda i: (i, 0)),
            pl.BlockSpec((3, n), lambda i: (0, 0)),
            pl.BlockSpec((1, 1), lambda i: (0, 0)),
            pl.BlockSpec((n, d_out), lambda i: (0, 0)),
        ],
        out_specs=pl.BlockSpec((b, d_out), lambda i: (i, 0)),
        out_shape=jax.ShapeDtypeStruct((n, d_out), jnp.float32),
    )(adj, s1c, aux, mx, h)
    return out
